# Initial kernel scaffold; baseline (speedup 1.0000x reference)
import jax
import jax.numpy as jnp
from jax.experimental import pallas as pl


def kernel(x, organism_index, norm_weight, embed_table):
    raise NotImplementedError("write your pallas kernel here")


# speedup vs baseline: 2.0217x; 2.0217x over previous
import functools
import math

import jax
import jax.numpy as jnp
import numpy as np
from jax.experimental import pallas as pl
from jax.experimental.pallas import tpu as pltpu

_EPS = 1e-5
_TILE = 128


def _gelu_exact(v):
    return 0.5 * v * (1.0 + jax.lax.erf(v * np.float32(1.0 / math.sqrt(2.0))))


def _stats_body(bi_ref, ti_ref, tj_ref, oi_ref, a_ref, b_ref, o_ref):
    p = pl.program_id(0)
    a = a_ref[...]
    bt = jnp.transpose(b_ref[...], (0, 2, 1, 3))
    sym = (a + bt) * 0.5
    contrib = jnp.sum(sym * sym, axis=(0, 1, 2)).reshape(1, -1)
    w = jnp.where(ti_ref[p] == tj_ref[p], 1.0, 2.0).astype(jnp.float32)

    @pl.when(p == 0)
    def _():
        o_ref[...] = jnp.zeros_like(o_ref)

    o_ref[...] += w * contrib


def _apply_body(bi_ref, ti_ref, tj_ref, oi_ref, a_ref, b_ref, scale_ref,
                emb_ref, o_ref):
    s = pl.program_id(1)
    c = a_ref.shape[-1]
    scale = scale_ref[...].reshape(1, 1, 1, c)
    emb = emb_ref[...].reshape(1, 1, 1, c)

    @pl.when(s == 0)
    def _():
        sym = (a_ref[...] + jnp.transpose(b_ref[...], (0, 2, 1, 3))) * 0.5
        o_ref[...] = _gelu_exact(sym * scale + emb)

    @pl.when(s == 1)
    def _():
        sym = (b_ref[...] + jnp.transpose(a_ref[...], (0, 2, 1, 3))) * 0.5
        o_ref[...] = _gelu_exact(sym * scale + emb)


def kernel(x, organism_index, norm_weight, embed_table):
    bsz, seq, seq2, ch = x.shape
    assert seq == seq2 and seq % _TILE == 0 and ch == _TILE
    nt = seq // _TILE
    pairs = [(i, j) for i in range(nt) for j in range(i, nt)]
    np_pairs = len(pairs)
    npairs = bsz * np_pairs

    bi = jnp.asarray(np.repeat(np.arange(bsz), np_pairs), jnp.int32)
    ti = jnp.asarray(np.tile([p[0] for p in pairs], bsz), jnp.int32)
    tj = jnp.asarray(np.tile([p[1] for p in pairs], bsz), jnp.int32)
    oi = jnp.asarray(organism_index, jnp.int32)

    tile_spec_a = pl.BlockSpec(
        (1, _TILE, _TILE, ch),
        lambda p, *refs: (refs[-4][p], refs[-3][p], refs[-2][p], 0))
    tile_spec_b = pl.BlockSpec(
        (1, _TILE, _TILE, ch),
        lambda p, *refs: (refs[-4][p], refs[-2][p], refs[-3][p], 0))

    sumsq = pl.pallas_call(
        _stats_body,
        grid_spec=pltpu.PrefetchScalarGridSpec(
            num_scalar_prefetch=4,
            grid=(npairs,),
            in_specs=[tile_spec_a, tile_spec_b],
            out_specs=pl.BlockSpec((1, ch), lambda p, *refs: (0, 0)),
        ),
        out_shape=jax.ShapeDtypeStruct((1, ch), jnp.float32),
        compiler_params=pltpu.CompilerParams(
            dimension_semantics=("arbitrary",)),
    )(bi, ti, tj, oi, x, x)

    n_total = bsz * seq * seq
    scale = (norm_weight * jax.lax.rsqrt(sumsq[0] / n_total + _EPS)).reshape(
        1, ch)

    def _in_a(p, s, bi_r, ti_r, tj_r, oi_r):
        return (bi_r[p], ti_r[p], tj_r[p], 0)

    def _in_b(p, s, bi_r, ti_r, tj_r, oi_r):
        return (bi_r[p], tj_r[p], ti_r[p], 0)

    def _out_map(p, s, bi_r, ti_r, tj_r, oi_r):
        return (bi_r[p], jnp.where(s == 0, ti_r[p], tj_r[p]),
                jnp.where(s == 0, tj_r[p], ti_r[p]), 0)

    def _emb_map(p, s, bi_r, ti_r, tj_r, oi_r):
        return (oi_r[bi_r[p]], 0, 0)

    embed_table_3d = embed_table.reshape(embed_table.shape[0], 1, ch)

    out = pl.pallas_call(
        _apply_body,
        grid_spec=pltpu.PrefetchScalarGridSpec(
            num_scalar_prefetch=4,
            grid=(npairs, 2),
            in_specs=[
                pl.BlockSpec((1, _TILE, _TILE, ch), _in_a),
                pl.BlockSpec((1, _TILE, _TILE, ch), _in_b),
                pl.BlockSpec((1, ch), lambda p, s, *refs: (0, 0)),
                pl.BlockSpec((1, 1, ch), _emb_map),
            ],
            out_specs=pl.BlockSpec((1, _TILE, _TILE, ch), _out_map),
        ),
        out_shape=jax.ShapeDtypeStruct(x.shape, jnp.float32),
        compiler_params=pltpu.CompilerParams(
            dimension_semantics=("arbitrary", "arbitrary")),
    )(bi, ti, tj, oi, x, x, scale, embed_table_3d)
    return out
